# Initial kernel scaffold; baseline (speedup 1.0000x reference)
import jax
import jax.numpy as jnp
from jax.experimental import pallas as pl


def kernel(x, edge_index, edge_attr, batch, params):
    raise NotImplementedError("write your pallas kernel here")


# speedup vs baseline: 2.2330x; 2.2330x over previous
import functools

import jax
import jax.numpy as jnp
from jax import lax
from jax.experimental import pallas as pl
from jax.experimental.pallas import tpu as pltpu
from jax.experimental.pallas import tpu_sc as plsc

N = 10000
E = 320000
HID = 128
OUT = 40
B = 8

NC = 2
NS = 16
L = 16
NW = NC * NS

C = 128
EP = ((E + NW * C - 1) // (NW * C)) * (NW * C)
EPT = EP // NW
NCHUNK = EPT // C
NP = 10112
RPW = NP // NS
DUMMY = N

_f32 = jnp.float32


def _ewb_body(ea_ref, we_ref, bm_ref, o0_ref, o1_ref, o2_ref):
    ea = ea_ref[...]
    outs = (o0_ref, o1_ref, o2_ref)
    for l in range(3):
        outs[l][...] = (
            jnp.dot(ea, we_ref[l], preferred_element_type=_f32) + bm_ref[l]
        )


_EWB_R = 4096


def _ewb_call(ea_pad, we_s, bm_s):
    grid = (EP // _EWB_R,)
    return pl.pallas_call(
        _ewb_body,
        grid=grid,
        in_specs=[
            pl.BlockSpec((_EWB_R, 8), lambda i: (i, 0)),
            pl.BlockSpec((3, 8, HID), lambda i: (0, 0, 0)),
            pl.BlockSpec((3, 1, HID), lambda i: (0, 0, 0)),
        ],
        out_specs=[
            pl.BlockSpec((_EWB_R, HID), lambda i: (i, 0)),
            pl.BlockSpec((_EWB_R, HID), lambda i: (i, 0)),
            pl.BlockSpec((_EWB_R, HID), lambda i: (i, 0)),
        ],
        out_shape=[jax.ShapeDtypeStruct((EP, HID), _f32)] * 3,
    )(ea_pad, we_s, bm_s)


def _transform_body(h_ref, wn_ref, ws_ref, hn_ref, hs_ref):
    h = h_ref[...]
    hn_ref[...] = jnp.dot(h, wn_ref[...], preferred_element_type=_f32)
    hs_ref[...] = jnp.dot(h, ws_ref[...], preferred_element_type=_f32)


def _transform_call(h, wn, ws):
    return pl.pallas_call(
        _transform_body,
        out_shape=[jax.ShapeDtypeStruct((N, HID), _f32)] * 2,
    )(h, wn, ws)


def _update_body(hs_ref, agg_ref, bu_ref, out_ref):
    a0 = agg_ref[pl.ds(0, N), :]
    a1 = agg_ref[pl.ds(NP, N), :]
    out_ref[...] = jnp.maximum(hs_ref[...] + a0 + a1 + bu_ref[...], 0.0)


def _update_call(hs, agg, bu):
    return pl.pallas_call(
        _update_body,
        out_shape=jax.ShapeDtypeStruct((N, HID), _f32),
    )(hs, agg, bu)


def _pool_body(h_ref, b_ref, wc1_ref, bc1_ref, wc2_ref, bc2_ref, out_ref):
    h = h_ref[...]
    bv = b_ref[...]
    neg = jnp.full_like(h, -jnp.inf)
    rows = [
        jnp.max(jnp.where(bv == g, h, neg), axis=0, keepdims=True)
        for g in range(B)
    ]
    pooled = jnp.concatenate(rows, axis=0)
    z = jnp.maximum(
        jnp.dot(pooled, wc1_ref[...], preferred_element_type=_f32)
        + bc1_ref[...], 0.0)
    out_ref[...] = (
        jnp.dot(z, wc2_ref[...], preferred_element_type=_f32) + bc2_ref[...]
    )


def _pool_call(h, batch2d, wc1, bc1, wc2, bc2):
    return pl.pallas_call(
        _pool_body,
        out_shape=jax.ShapeDtypeStruct((B, HID), _f32),
    )(h, batch2d, wc1, bc1, wc2, bc2)


_sc_mesh = plsc.VectorSubcoreMesh(
    core_axis_name="c", subcore_axis_name="s", num_cores=NC, num_subcores=NS)


@functools.partial(
    pl.kernel,
    out_type=jax.ShapeDtypeStruct((NC * NP, HID), _f32),
    mesh=_sc_mesh,
    scratch_types=[
        pltpu.VMEM((C,), jnp.int32),
        pltpu.VMEM((C,), jnp.int32),
        pltpu.VMEM((C, HID), _f32),
        pltpu.VMEM((C, HID), _f32),
        pltpu.VMEM_SHARED((NP, HID), _f32),
        pltpu.SemaphoreType.DMA,
    ],
)
def _edge_sc(hn_hbm, ewb_hbm, src_hbm, dst_hbm, zeros_hbm, out_hbm,
             sidx, didx, rows, ews, acc, sem):
    cid = lax.axis_index("c")
    sid = lax.axis_index("s")
    wid = sid * NC + cid
    base = wid * EPT

    pltpu.sync_copy(zeros_hbm.at[pl.ds(sid * RPW, RPW)],
                    acc.at[pl.ds(sid * RPW, RPW)])
    plsc.subcore_barrier()

    def chunk_body(i, _):
        off = base + i * C
        pltpu.sync_copy(src_hbm.at[pl.ds(off, C)], sidx)
        pltpu.sync_copy(dst_hbm.at[pl.ds(off, C)], didx)
        pltpu.sync_copy(ewb_hbm.at[pl.ds(off, C)], ews)
        pltpu.async_copy(hn_hbm.at[sidx], rows, sem).wait()

        def relu_body(e, _):
            for j in range(HID // L):
                sl = pl.ds(j * L, L)
                rows[e, sl] = jnp.maximum(rows[e, sl] + ews[e, sl], 0.0)
            return 0

        lax.fori_loop(0, C, relu_body, 0)
        pltpu.sync_copy(rows, acc.at[didx], add=True)
        return 0

    lax.fori_loop(0, NCHUNK, chunk_body, 0)
    plsc.subcore_barrier()
    pltpu.sync_copy(acc.at[pl.ds(sid * RPW, RPW)],
                    out_hbm.at[pl.ds(cid * NP + sid * RPW, RPW)])


def kernel(x, edge_index, edge_attr, batch, params):
    src = edge_index[0]
    dst = edge_index[1]
    srcp = jnp.pad(src, (0, EP - E))
    dstp = jnp.pad(dst, (0, EP - E), constant_values=DUMMY)
    ea_pad = jnp.pad(edge_attr, ((0, EP - E), (0, 8 - edge_attr.shape[1])))

    layers = params["layers"]
    we_s = jnp.stack([
        jnp.pad(lp["We"], ((0, 8 - lp["We"].shape[0]), (0, 0)))
        for lp in layers])
    bm_s = jnp.stack([lp["bm"][None, :] for lp in layers])
    ewbs = _ewb_call(ea_pad, we_s, bm_s)

    zeros = jnp.zeros((NP, HID), _f32)
    batch2d = batch[:, None]

    h = jnp.pad(x, ((0, 0), (0, 8 - x.shape[1])))
    for l, lp in enumerate(layers):
        d = h.shape[1]
        wn = jnp.pad(lp["Wn"], ((0, d - lp["Wn"].shape[0]), (0, 0)))
        ws = jnp.pad(lp["Ws"], ((0, d - lp["Ws"].shape[0]), (0, 0)))
        hn, hs = _transform_call(h, wn, ws)
        agg = _edge_sc(hn, ewbs[l], srcp, dstp, zeros)
        h = _update_call(hs, agg, lp["bu"][None, :])

    wc2 = jnp.pad(params["Wc2"], ((0, 0), (0, HID - OUT)))
    bc2 = jnp.pad(params["bc2"], (0, HID - OUT))[None, :]
    logits = _pool_call(h, batch2d, params["Wc1"], params["bc1"][None, :],
                        wc2, bc2)
    return logits[:, :OUT]
